# baseline (speedup 1.0000x reference)
import functools

import jax
import jax.numpy as jnp
from jax import lax
from jax.experimental import pallas as pl
from jax.experimental.pallas import tpu as pltpu
from jax.experimental.pallas import tpu_sc as plsc

D = 128
DC = D // 16
B = 16384
S = 64
NC, NS = 2, 16
NW = NC * NS
QPW = B // NW
CH = 64
NCHUNK = QPW // CH
EPS2 = 1e-16

_GATHER_DNUMS = lax.GatherDimensionNumbers(
    offset_dims=(), collapsed_slice_dims=(0,), start_index_map=(0,))


def _lane_shuffle(v, idx):
    return lax.gather(v, idx[:, None], dimension_numbers=_GATHER_DNUMS,
                      slice_sizes=(1,),
                      mode=lax.GatherScatterMode.PROMISE_IN_BOUNDS)


def _hsum(v):
    lane = lax.iota(jnp.int32, 16)
    for off in (8, 4, 2, 1):
        v = v + _lane_shuffle(v, lane ^ off)
    return v


def _rsqrt(x):
    i = lax.bitcast_convert_type(x, jnp.int32)
    i = jnp.int32(0x5F3759DF) - (i >> 1)
    y = lax.bitcast_convert_type(i, jnp.float32)
    for _ in range(3):
        y = y * (1.5 - 0.5 * x * y * y)
    return y


def _sc_body(table, qflat, supflat, out, sup_idx_v, sup_rows_v,
             idx_a, idx_b, rows_a, rows_b, out_v, ssem, sem_a, sem_b):
    wid = lax.axis_index("s") * NC + lax.axis_index("c")
    base = wid * QPW

    bufs = [(idx_a, rows_a, sem_a), (idx_b, rows_b, sem_b)]

    def fire(c, buf):
        idx_v, rows_v, sem = buf
        pltpu.sync_copy(qflat.at[pl.ds(2 * (base + c * CH), 2 * CH)], idx_v)
        return pltpu.async_copy(table.at[idx_v], rows_v, sem)

    pltpu.sync_copy(supflat, sup_idx_v)
    sup_cp = pltpu.async_copy(table.at[sup_idx_v], sup_rows_v, ssem)
    pending = fire(0, bufs[0])
    sup_cp.wait()

    zeros = jnp.zeros((16,), jnp.float32)

    def sup_body(j, accs):
        new = []
        for k in range(DC):
            new.append(accs[k] + sup_rows_v[2 * j, pl.ds(k * 16, 16)])
        for k in range(DC):
            new.append(accs[DC + k] + sup_rows_v[2 * j + 1, pl.ds(k * 16, 16)])
        return tuple(new)

    accs = lax.fori_loop(0, S, sup_body, (zeros,) * (2 * DC))
    m = [a * (1.0 / S) for a in accs]

    msq = zeros
    for k in range(2 * DC):
        msq = msq + m[k] * m[k]
    rs_s = _rsqrt(jnp.maximum(_hsum(msq), EPS2))

    lane = lax.iota(jnp.int32, 16)

    def compute(c, buf):
        _, rows_v, _ = buf

        def blk_body(j16, _):
            numvec = zeros
            sqvec = zeros
            for l in range(16):
                j = 2 * (j16 * 16 + l)
                num = zeros
                sq = zeros
                for k in range(DC):
                    r0 = rows_v[j, pl.ds(k * 16, 16)]
                    r1 = rows_v[j + 1, pl.ds(k * 16, 16)]
                    num = num + r0 * m[k] + r1 * m[DC + k]
                    sq = sq + r0 * r0 + r1 * r1
                sel = lane == l
                numvec = jnp.where(sel, _hsum(num), numvec)
                sqvec = jnp.where(sel, _hsum(sq), sqvec)
            res = numvec * _rsqrt(jnp.maximum(sqvec, EPS2)) * rs_s
            out_v[pl.ds(j16 * 16, 16)] = res
            return 0

        lax.fori_loop(0, CH // 16, blk_body, 0)
        pltpu.sync_copy(out_v, out.at[pl.ds(base + c * CH, CH)])

    for c in range(NCHUNK):
        nxt = fire(c + 1, bufs[(c + 1) % 2]) if c + 1 < NCHUNK else None
        pending.wait()
        compute(c, bufs[c % 2])
        pending = nxt


@functools.partial(jax.jit, donate_argnums=())
def _run(table, qflat, supflat):
    mesh = plsc.VectorSubcoreMesh(core_axis_name="c", subcore_axis_name="s",
                                  num_cores=NC, num_subcores=NS)
    return pl.kernel(
        _sc_body,
        out_type=jax.ShapeDtypeStruct((B,), jnp.float32),
        mesh=mesh,
        scratch_types=[
            pltpu.VMEM((2 * S,), jnp.int32),
            pltpu.VMEM((2 * S, D), jnp.float32),
            pltpu.VMEM((2 * CH,), jnp.int32),
            pltpu.VMEM((2 * CH,), jnp.int32),
            pltpu.VMEM((2 * CH, D), jnp.float32),
            pltpu.VMEM((2 * CH, D), jnp.float32),
            pltpu.VMEM((CH,), jnp.float32),
            pltpu.SemaphoreType.DMA,
            pltpu.SemaphoreType.DMA,
            pltpu.SemaphoreType.DMA,
        ],
    )(table, qflat, supflat)


def kernel(query, support, symbol_emb):
    qflat = query.astype(jnp.int32).reshape(2 * B)
    supflat = support.astype(jnp.int32).reshape(2 * S)
    return _run(symbol_emb, qflat, supflat)


# speedup vs baseline: 1.0631x; 1.0631x over previous
import functools

import jax
import jax.numpy as jnp
from jax import lax
from jax.experimental import pallas as pl
from jax.experimental.pallas import tpu as pltpu
from jax.experimental.pallas import tpu_sc as plsc

D = 128
DC = D // 16
B = 16384
S = 64
NC, NS = 2, 16
NW = NC * NS
QPW = B // NW
CH = 128
NCHUNK = QPW // CH
EPS2 = 1e-16

_GATHER_DNUMS = lax.GatherDimensionNumbers(
    offset_dims=(), collapsed_slice_dims=(0,), start_index_map=(0,))


def _lane_shuffle(v, idx):
    return lax.gather(v, idx[:, None], dimension_numbers=_GATHER_DNUMS,
                      slice_sizes=(1,),
                      mode=lax.GatherScatterMode.PROMISE_IN_BOUNDS)


def _hsum(v):
    lane = lax.iota(jnp.int32, 16)
    for off in (8, 4, 2, 1):
        v = v + _lane_shuffle(v, lane ^ off)
    return v


def _rsqrt(x):
    i = lax.bitcast_convert_type(x, jnp.int32)
    i = jnp.int32(0x5F3759DF) - (i >> 1)
    y = lax.bitcast_convert_type(i, jnp.float32)
    for _ in range(3):
        y = y * (1.5 - 0.5 * x * y * y)
    return y


def _sc_body(table, qflat, supflat, out, sup_idx_v, sup_rows_v,
             idx_a, idx_b, rows_a, rows_b, out_v, ssem, sem_a, sem_b):
    wid = lax.axis_index("s") * NC + lax.axis_index("c")
    base = wid * QPW

    bufs = [(idx_a, rows_a, sem_a), (idx_b, rows_b, sem_b)]

    def fire(c, buf):
        (i0, i1), rows_v, sem = buf
        start = 2 * (base + c * CH)
        pltpu.sync_copy(qflat.at[pl.ds(start, CH)], i0)
        pltpu.sync_copy(qflat.at[pl.ds(start + CH, CH)], i1)
        cp0 = pltpu.async_copy(table.at[i0], rows_v.at[pl.ds(0, CH)], sem)
        cp1 = pltpu.async_copy(table.at[i1], rows_v.at[pl.ds(CH, CH)], sem)
        return cp0, cp1

    pltpu.sync_copy(supflat, sup_idx_v)
    sup_cp = pltpu.async_copy(table.at[sup_idx_v], sup_rows_v, ssem)
    pending = fire(0, bufs[0])
    sup_cp.wait()
    del sup_cp

    zeros = jnp.zeros((16,), jnp.float32)

    def sup_body(j, accs):
        new = []
        for k in range(DC):
            new.append(accs[k] + sup_rows_v[2 * j, pl.ds(k * 16, 16)])
        for k in range(DC):
            new.append(accs[DC + k] + sup_rows_v[2 * j + 1, pl.ds(k * 16, 16)])
        return tuple(new)

    accs = lax.fori_loop(0, S, sup_body, (zeros,) * (2 * DC))
    m = [a * (1.0 / S) for a in accs]

    msq = zeros
    for k in range(2 * DC):
        msq = msq + m[k] * m[k]
    rs_s = _rsqrt(jnp.maximum(_hsum(msq), EPS2))

    lane = lax.iota(jnp.int32, 16)

    def compute(c, buf):
        _, rows_v, _ = buf

        def blk_body(j16, _):
            numvec = zeros
            sqvec = zeros
            for l in range(16):
                j = 2 * (j16 * 16 + l)
                num = zeros
                sq = zeros
                for k in range(DC):
                    r0 = rows_v[j, pl.ds(k * 16, 16)]
                    r1 = rows_v[j + 1, pl.ds(k * 16, 16)]
                    num = num + r0 * m[k] + r1 * m[DC + k]
                    sq = sq + r0 * r0 + r1 * r1
                sel = lane == l
                numvec = jnp.where(sel, _hsum(num), numvec)
                sqvec = jnp.where(sel, _hsum(sq), sqvec)
            res = numvec * _rsqrt(jnp.maximum(sqvec, EPS2)) * rs_s
            out_v[pl.ds(j16 * 16, 16)] = res
            return 0

        lax.fori_loop(0, CH // 16, blk_body, 0)
        pltpu.sync_copy(out_v, out.at[pl.ds(base + c * CH, CH)])

    for c in range(NCHUNK):
        nxt = fire(c + 1, bufs[(c + 1) % 2]) if c + 1 < NCHUNK else None
        for cp in pending:
            cp.wait()
        compute(c, bufs[c % 2])
        pending = nxt


@functools.partial(jax.jit, donate_argnums=())
def _run(table, qflat, supflat):
    mesh = plsc.VectorSubcoreMesh(core_axis_name="c", subcore_axis_name="s",
                                  num_cores=NC, num_subcores=NS)
    return pl.kernel(
        _sc_body,
        out_type=jax.ShapeDtypeStruct((B,), jnp.float32),
        mesh=mesh,
        scratch_types=[
            pltpu.VMEM((2 * S,), jnp.int32),
            pltpu.VMEM((2 * S, D), jnp.float32),
            (pltpu.VMEM((CH,), jnp.int32),
             pltpu.VMEM((CH,), jnp.int32)),
            (pltpu.VMEM((CH,), jnp.int32),
             pltpu.VMEM((CH,), jnp.int32)),
            pltpu.VMEM((2 * CH, D), jnp.float32),
            pltpu.VMEM((2 * CH, D), jnp.float32),
            pltpu.VMEM((CH,), jnp.float32),
            pltpu.SemaphoreType.DMA,
            pltpu.SemaphoreType.DMA,
            pltpu.SemaphoreType.DMA,
        ],
    )(table, qflat, supflat)


def kernel(query, support, symbol_emb):
    qflat = query.astype(jnp.int32).reshape(2 * B)
    supflat = support.astype(jnp.int32).reshape(2 * S)
    return _run(symbol_emb, qflat, supflat)
